# initial kernel scaffold (unmeasured)
import jax
import jax.numpy as jnp
from jax import lax
from jax.experimental import pallas as pl
from jax.experimental.pallas import tpu as pltpu

N_DEV = 4
SQ = 2048
D_MODEL = 1024
DH = 128
HEADS = 8
CHUNK = HEADS * DH
SCALE = 0.08838834764831843


def _rope(t, cosf, sinf):
    lane = lax.broadcasted_iota(jnp.int32, t.shape, 1)
    even = (lane % 2) == 0
    t_next = pltpu.roll(t, -1, 1)
    t_prev = pltpu.roll(t, 1, 1)
    t_rot = jnp.where(even, -t_next, t_prev)
    return t * cosf + t_rot * sinf


def _body(x_ref, cos_ref, sin_ref, wq_ref, wk_ref, wv_ref, wo_ref, out_ref,
          comm_ref, wvmem_ref, copy_sems, load_sems, send_sems, recv_sems):
    my_pos = lax.axis_index("i")
    left = (my_pos + N_DEV - 1) % N_DEV
    right = (my_pos + 1) % N_DEV

    barrier_sem = pltpu.get_barrier_semaphore()
    for nbr in (left, right):
        pl.semaphore_signal(barrier_sem, inc=1, device_id=(nbr,),
                            device_id_type=pl.DeviceIdType.MESH)
    pl.semaphore_wait(barrier_sem, 2)

    copies = []
    for j, ref in enumerate((wq_ref, wk_ref, wv_ref, wo_ref)):
        cp = pltpu.make_async_copy(ref, comm_ref.at[0, j], copy_sems.at[j])
        cp.start()
        copies.append(cp)
    for cp in copies:
        cp.wait()

    x2d = x_ref[0, :, :]
    cosf = jnp.concatenate([cos_ref[:, :]] * HEADS, axis=1)
    sinf = jnp.concatenate([sin_ref[:, :]] * HEADS, axis=1)

    sends = []
    for c in range(N_DEV):
        if c > 0:
            recv = pltpu.make_async_remote_copy(
                src_ref=comm_ref.at[c], dst_ref=comm_ref.at[c],
                send_sem=send_sems.at[c - 1], recv_sem=recv_sems.at[c - 1],
                device_id=(left,), device_id_type=pl.DeviceIdType.MESH)
            recv.wait_recv()
        if c < N_DEV - 1:
            snd = pltpu.make_async_remote_copy(
                src_ref=comm_ref.at[c], dst_ref=comm_ref.at[c + 1],
                send_sem=send_sems.at[c], recv_sem=recv_sems.at[c],
                device_id=(right,), device_id_type=pl.DeviceIdType.MESH)
            snd.start()
            sends.append(snd)

        slot = c % 2
        ld = pltpu.make_async_copy(comm_ref.at[c], wvmem_ref.at[slot],
                                   load_sems.at[slot])
        ld.start()
        ld.wait()

        wq = wvmem_ref[slot, 0]
        wk = wvmem_ref[slot, 1]
        wv = wvmem_ref[slot, 2]
        wo = wvmem_ref[slot, 3]

        q = jnp.dot(x2d, wq, preferred_element_type=jnp.float32)
        k = jnp.dot(x2d, wk, preferred_element_type=jnp.float32)
        v = jnp.dot(x2d, wv, preferred_element_type=jnp.float32)
        q = _rope(q, cosf, sinf) * SCALE
        k = _rope(k, cosf, sinf)

        ctx_parts = []
        for h in range(HEADS):
            sl = slice(h * DH, (h + 1) * DH)
            qh, kh, vh = q[:, sl], k[:, sl], v[:, sl]
            s = lax.dot_general(qh, kh, (((1,), (1,)), ((), ())),
                                preferred_element_type=jnp.float32)
            m = jnp.max(s, axis=1, keepdims=True)
            p = jnp.exp(s - m)
            p = p / jnp.sum(p, axis=1, keepdims=True)
            ctx_parts.append(jnp.dot(p, vh, preferred_element_type=jnp.float32))
        ctx = jnp.concatenate(ctx_parts, axis=1)
        outc = jnp.dot(ctx, wo, preferred_element_type=jnp.float32)
        if c == 0:
            out_ref[0, :, :] = outc
        else:
            out_ref[0, :, :] = out_ref[0, :, :] + outc

    for snd in sends:
        snd.wait_send()


def kernel(x, Wq, Wk, Wv, Wo):
    pos = jnp.arange(SQ, dtype=jnp.float32)[:, None]
    inv = 1.0 / (10000.0 ** (jnp.arange(0, DH, 2, dtype=jnp.float32) / DH))
    ang = pos * inv[None, :]
    cos = jnp.repeat(jnp.cos(ang), 2, axis=1)
    sin = jnp.repeat(jnp.sin(ang), 2, axis=1)

    vmem = pl.BlockSpec(memory_space=pltpu.MemorySpace.VMEM)
    hbm = pl.BlockSpec(memory_space=pl.ANY)
    return pl.pallas_call(
        _body,
        out_shape=jax.ShapeDtypeStruct((1, SQ, D_MODEL), jnp.float32),
        in_specs=[vmem, vmem, vmem, hbm, hbm, hbm, hbm],
        out_specs=vmem,
        scratch_shapes=[
            pltpu.MemorySpace.HBM((N_DEV, 4, D_MODEL, CHUNK), jnp.float32),
            pltpu.VMEM((2, 4, D_MODEL, CHUNK), jnp.float32),
            pltpu.SemaphoreType.DMA((4,)),
            pltpu.SemaphoreType.DMA((2,)),
            pltpu.SemaphoreType.DMA((N_DEV - 1,)),
            pltpu.SemaphoreType.DMA((N_DEV - 1,)),
        ],
        compiler_params=pltpu.CompilerParams(
            collective_id=0,
            vmem_limit_bytes=128 * 1024 * 1024,
        ),
    )(x, cos, sin, Wq, Wk, Wv, Wo)


# baseline (device time: 1119757 ns/iter reference)
import jax
import jax.numpy as jnp
from jax import lax
from jax.experimental import pallas as pl
from jax.experimental.pallas import tpu as pltpu

N_DEV = 4
SQ = 2048
D_MODEL = 1024
DH = 128
HEADS = 8
RBLK = 512
SCALE = 0.08838834764831843
F32 = jnp.float32


def _rope(t, cos, sin):
    lane = lax.broadcasted_iota(jnp.int32, t.shape, 1)
    even = (lane % 2) == 0
    t_next = pltpu.roll(t, t.shape[1] - 1, 1)
    t_prev = pltpu.roll(t, 1, 1)
    return t * cos + jnp.where(even, -t_next, t_prev) * sin


def _body(x_ref, cos_ref, sin_ref, wqkv_ref, wo_ref,
          out_ref, comm_qkv, comm_wo,
          wstage, wostage, copy_sems, stage_sems,
          qkv_send, qkv_recv, wo_send, wo_recv):
    c = pl.program_id(0)
    h = pl.program_id(1)
    my_pos = lax.axis_index("i")
    left = (my_pos + N_DEV - 1) % N_DEV
    right = (my_pos + 1) % N_DEV

    @pl.when(jnp.logical_and(c == 0, h == 0))
    def _pack_own():
        cp1 = pltpu.make_async_copy(wqkv_ref, comm_qkv.at[0], copy_sems.at[0])
        cp2 = pltpu.make_async_copy(wo_ref, comm_wo.at[0], copy_sems.at[1])
        cp1.start()
        cp2.start()
        cp1.wait()
        cp2.wait()

    @pl.when(jnp.logical_and(c > 0, h == 0))
    def _recv_and_drain():
        cm = jnp.maximum(c, 1)
        r1 = pltpu.make_async_remote_copy(
            src_ref=comm_qkv.at[cm], dst_ref=comm_qkv.at[cm],
            send_sem=qkv_send.at[cm - 1], recv_sem=qkv_recv.at[cm - 1],
            device_id=(left,), device_id_type=pl.DeviceIdType.MESH)
        r2 = pltpu.make_async_remote_copy(
            src_ref=comm_wo.at[cm], dst_ref=comm_wo.at[cm],
            send_sem=wo_send.at[cm - 1], recv_sem=wo_recv.at[cm - 1],
            device_id=(left,), device_id_type=pl.DeviceIdType.MESH)
        r1.wait_recv()
        r2.wait_recv()
        s1 = pltpu.make_async_remote_copy(
            src_ref=comm_qkv.at[cm - 1], dst_ref=comm_qkv.at[cm],
            send_sem=qkv_send.at[cm - 1], recv_sem=qkv_recv.at[cm - 1],
            device_id=(right,), device_id_type=pl.DeviceIdType.MESH)
        s2 = pltpu.make_async_remote_copy(
            src_ref=comm_wo.at[cm - 1], dst_ref=comm_wo.at[cm],
            send_sem=wo_send.at[cm - 1], recv_sem=wo_recv.at[cm - 1],
            device_id=(right,), device_id_type=pl.DeviceIdType.MESH)
        s1.wait_send()
        s2.wait_send()

    @pl.when(jnp.logical_and(c < N_DEV - 1, h == 0))
    def _forward():
        cm = jnp.minimum(c, N_DEV - 2)
        s1 = pltpu.make_async_remote_copy(
            src_ref=comm_qkv.at[cm], dst_ref=comm_qkv.at[cm + 1],
            send_sem=qkv_send.at[cm], recv_sem=qkv_recv.at[cm],
            device_id=(right,), device_id_type=pl.DeviceIdType.MESH)
        s2 = pltpu.make_async_remote_copy(
            src_ref=comm_wo.at[cm], dst_ref=comm_wo.at[cm + 1],
            send_sem=wo_send.at[cm], recv_sem=wo_recv.at[cm],
            device_id=(right,), device_id_type=pl.DeviceIdType.MESH)
        s1.start()
        s2.start()

    stages = []
    for j in range(3):
        st = pltpu.make_async_copy(comm_qkv.at[c, j, h], wstage.at[j],
                                   stage_sems.at[j])
        st.start()
        stages.append(st)
    st = pltpu.make_async_copy(comm_wo.at[c, h], wostage, stage_sems.at[3])
    st.start()
    stages.append(st)
    for st in stages:
        st.wait()

    cos = cos_ref[:, :]
    sin = sin_ref[:, :]
    wq = wstage[0]
    wk = wstage[1]
    wv = wstage[2]
    woh = wostage[:, :]
    x2d = x_ref[0]

    kh = _rope(jnp.dot(x2d, wk, preferred_element_type=F32), cos, sin)
    vh = jnp.dot(x2d, wv, preferred_element_type=F32)

    first = jnp.logical_and(c == 0, h == 0)
    for r in range(SQ // RBLK):
        rows = slice(r * RBLK, (r + 1) * RBLK)
        xr = x_ref[0, rows, :]
        qr = _rope(jnp.dot(xr, wq, preferred_element_type=F32),
                   cos[rows, :], sin[rows, :]) * SCALE
        s = lax.dot_general(qr, kh, (((1,), (1,)), ((), ())),
                            preferred_element_type=F32)
        m = jnp.max(s, axis=1, keepdims=True)
        p = jnp.exp(s - m)
        p = p / jnp.sum(p, axis=1, keepdims=True)
        ctx = jnp.dot(p, vh, preferred_element_type=F32)
        outc = jnp.dot(ctx, woh, preferred_element_type=F32)

        @pl.when(first)
        def _init():
            out_ref[0, rows, :] = outc

        @pl.when(jnp.logical_not(first))
        def _acc():
            out_ref[0, rows, :] = out_ref[0, rows, :] + outc


def kernel(x, Wq, Wk, Wv, Wo):
    wqkv = jnp.stack([
        W.reshape(D_MODEL, HEADS, DH).transpose(1, 0, 2)
        for W in (Wq, Wk, Wv)
    ])
    wo_p = Wo.reshape(HEADS, DH, D_MODEL)

    pos = jnp.arange(SQ, dtype=F32)[:, None]
    inv = 1.0 / (10000.0 ** (jnp.arange(0, DH, 2, dtype=F32) / DH))
    ang = pos * inv[None, :]
    cos = jnp.repeat(jnp.cos(ang), 2, axis=1)
    sin = jnp.repeat(jnp.sin(ang), 2, axis=1)

    vmem = pl.BlockSpec(memory_space=pltpu.MemorySpace.VMEM)
    hbm = pl.BlockSpec(memory_space=pl.ANY)
    out, _, _ = pl.pallas_call(
        _body,
        grid=(N_DEV, HEADS),
        out_shape=[
            jax.ShapeDtypeStruct((1, SQ, D_MODEL), F32),
            jax.ShapeDtypeStruct((N_DEV, 3, HEADS, D_MODEL, DH), F32),
            jax.ShapeDtypeStruct((N_DEV, HEADS, DH, D_MODEL), F32),
        ],
        in_specs=[vmem, vmem, vmem, hbm, hbm],
        out_specs=[vmem, hbm, hbm],
        scratch_shapes=[
            pltpu.VMEM((3, D_MODEL, DH), F32),
            pltpu.VMEM((DH, D_MODEL), F32),
            pltpu.SemaphoreType.DMA((2,)),
            pltpu.SemaphoreType.DMA((4,)),
            pltpu.SemaphoreType.DMA((N_DEV - 1,)),
            pltpu.SemaphoreType.DMA((N_DEV - 1,)),
            pltpu.SemaphoreType.DMA((N_DEV - 1,)),
            pltpu.SemaphoreType.DMA((N_DEV - 1,)),
        ],
        compiler_params=pltpu.CompilerParams(
            dimension_semantics=("arbitrary", "arbitrary"),
        ),
    )(x, cos, sin, wqkv, wo_p)
    return out
